# Initial kernel scaffold; baseline (speedup 1.0000x reference)
import jax
import jax.numpy as jnp
from jax.experimental import pallas as pl


def kernel(yhat, y):
    raise NotImplementedError("write your pallas kernel here")


# speedup vs baseline: 14.4510x; 14.4510x over previous
import dataclasses
import functools

import jax
import jax.numpy as jnp
from jax import lax
from jax.experimental import pallas as pl
from jax.experimental.pallas import tpu as pltpu
from jax.experimental.pallas import tpu_sc as plsc

N = 128 * 32768
K = int(N * 0.1)
L = 16
NW = 32
BLK = 8192
GRID = N // BLK

_mesh = functools.partial(
    plsc.VectorSubcoreMesh, core_axis_name="core", subcore_axis_name="subcore"
)


def _cparams():
    cp = pltpu.CompilerParams()
    if "needs_layout_passes" in pltpu.CompilerParams.__dataclass_fields__:
        cp = dataclasses.replace(cp, needs_layout_passes=False)
    return cp


def _bcast(x, dtype=jnp.int32):
    return lax.broadcast_in_dim(jnp.asarray(x, dtype), (L,), ())


def _zero_hist(hist_ref, nbins):
    zeros = jnp.zeros((L,), jnp.int32)

    @pl.loop(0, nbins, step=L)
    def _(i):
        hist_ref[pl.ds(i, L)] = zeros


def _reduce_hist(h_hbm, tmp_ref, hsum_ref, nbins):
    pltpu.sync_copy(h_hbm, tmp_ref)

    @pl.loop(0, nbins, step=L)
    def _(c):
        def body(t, acc):
            return acc + tmp_ref[t, pl.ds(c, L)]

        hsum_ref[pl.ds(c, L)] = lax.fori_loop(
            0, NW, body, jnp.zeros((L,), jnp.int32)
        )


def _find(hsum_ref, r, nbins):
    nchunks = nbins // L

    def body(i, carry):
        s, csel, hsel, sbefore = carry
        c = nchunks - 1 - i
        h = hsum_ref[pl.ds(c * L, L)]
        t = jnp.sum(h)
        hit = jnp.logical_and(s < r, s + t >= r)
        hitv = lax.broadcast_in_dim(hit, (L,), ())
        csel = jnp.where(hit, c, csel)
        hsel = jnp.where(hitv, h, hsel)
        sbefore = jnp.where(hit, s, sbefore)
        return s + t, csel, hsel, sbefore

    zero = jnp.asarray(0, jnp.int32)
    _, csel, hsel, sbefore = lax.fori_loop(
        0, nchunks, body, (zero, zero, jnp.zeros((L,), jnp.int32), zero)
    )
    cnt_desc = lax.rev(hsel, (0,))
    cum = jnp.cumsum(cnt_desc)
    r_in = r - sbefore
    i_star = jnp.sum((cum < r_in).astype(jnp.int32))
    at = lax.iota(jnp.int32, L) == lax.broadcast_in_dim(i_star, (L,), ())
    cnt_at = jnp.sum(jnp.where(at, cnt_desc, 0))
    cum_before = jnp.sum(jnp.where(at, cum, 0)) - cnt_at
    bucket = csel * L + (L - 1 - i_star)
    return bucket, r_in - cum_before


_LEVELS = ((23, 256), (15, 256), (7, 256), (0, 128))


def _threshold_prefix(h_hbms, tmp_refs, hsum_ref):
    r = jnp.asarray(K, jnp.int32)
    prefix = jnp.asarray(0, jnp.int32)
    for (_, nb), h_hbm, tmp_ref in zip(_LEVELS, h_hbms, tmp_refs):
        _reduce_hist(h_hbm, tmp_ref, hsum_ref, nb)
        b, r = _find(hsum_ref, r, nb)
        prefix = prefix * nb + b
    return prefix, r


def _wid():
    return lax.axis_index("subcore") * 2 + lax.axis_index("core")


def _scan_specs():
    return [pl.BlockSpec((BLK,), lambda i: (i,))]


_PIPE = dict(
    grid=(GRID,),
    core_axis_name=("core", "subcore"),
    dimension_semantics=(pltpu.PARALLEL,),
)


def _hist_scratch(nlevels):
    return [pltpu.VMEM((NW, nb), jnp.int32) for _, nb in _LEVELS[:nlevels]] + [
        pltpu.VMEM((256,), jnp.int32)
    ]


def _pass_a(yhat, y):
    @functools.partial(
        pl.kernel,
        out_type=(
            jax.ShapeDtypeStruct((N,), jnp.float32),
            jax.ShapeDtypeStruct((NW, 256), jnp.int32),
        ),
        mesh=_mesh(),
        scratch_types=[pltpu.VMEM((256,), jnp.int32)],
        compiler_params=_cparams(),
    )
    def k(a_hbm, b_hbm, mse_hbm, h_hbm, hist_ref):
        _zero_hist(hist_ref, 256)
        ones = jnp.ones((L,), jnp.int32)
        sh = _bcast(23)

        def body(a_ref, b_ref, m_ref):
            @pl.loop(0, BLK, step=L)
            def _(i):
                d = a_ref[pl.ds(i, L)] - b_ref[pl.ds(i, L)]
                m = d * d
                m_ref[pl.ds(i, L)] = m
                idx = lax.shift_right_logical(plsc.bitcast(m, jnp.int32), sh)
                plsc.addupdate_scatter(hist_ref, [idx], ones)

        pltpu.emit_pipeline(
            body, in_specs=_scan_specs() * 2, out_specs=_scan_specs(), **_PIPE
        )(a_hbm, b_hbm, mse_hbm)
        pltpu.sync_copy(hist_ref, h_hbm.at[_wid()])

    return k(yhat, y)


def _hist_pass(mse, hists):
    lvl = len(hists)
    shift, nbins = _LEVELS[lvl]

    @functools.partial(
        pl.kernel,
        out_type=jax.ShapeDtypeStruct((NW, nbins), jnp.int32),
        mesh=_mesh(),
        scratch_types=_hist_scratch(lvl) + [pltpu.VMEM((nbins,), jnp.int32)],
        compiler_params=_cparams(),
    )
    def k(mse_hbm, *refs):
        h_hbms = refs[:lvl]
        out_hbm = refs[lvl]
        tmp_refs = refs[lvl + 1 : 2 * lvl + 1]
        hsum_ref = refs[2 * lvl + 1]
        hist_ref = refs[2 * lvl + 2]

        prefix, _ = _threshold_prefix(h_hbms, tmp_refs, hsum_ref)

        _zero_hist(hist_ref, nbins)
        ones = jnp.ones((L,), jnp.int32)
        shv = _bcast(shift)
        nbits = 8 if nbins == 256 else 7
        cshv = _bcast(shift + nbits)
        maskv = _bcast(nbins - 1)
        prefv = lax.broadcast_in_dim(prefix, (L,), ())

        def body(m_ref):
            @pl.loop(0, BLK, step=L)
            def _(i):
                pat = plsc.bitcast(m_ref[pl.ds(i, L)], jnp.int32)
                cond = lax.shift_right_logical(pat, cshv) == prefv
                idx = jnp.bitwise_and(lax.shift_right_logical(pat, shv), maskv)
                plsc.addupdate_scatter(hist_ref, [idx], ones, mask=cond)

        pltpu.emit_pipeline(body, in_specs=_scan_specs(), out_specs=[], **_PIPE)(
            mse_hbm
        )
        pltpu.sync_copy(hist_ref, out_hbm.at[_wid()])

    return k(mse, *hists)


def _pass_out(mse, hists):
    @functools.partial(
        pl.kernel,
        out_type=jax.ShapeDtypeStruct((N,), jnp.float32),
        mesh=_mesh(),
        scratch_types=_hist_scratch(4),
        compiler_params=_cparams(),
    )
    def k(mse_hbm, h1, h2, h3, h4, out_hbm, t1, t2, t3, t4, hsum_ref):
        thresh, _ = _threshold_prefix((h1, h2, h3, h4), (t1, t2, t3, t4), hsum_ref)

        tv = lax.broadcast_in_dim(thresh, (L,), ())
        ten = jnp.full((L,), 10.0, jnp.float32)
        zf = jnp.zeros((L,), jnp.float32)

        def body(m_ref, o_ref):
            @pl.loop(0, BLK, step=L)
            def _(i):
                m = m_ref[pl.ds(i, L)]
                sel = plsc.bitcast(m, jnp.int32) >= tv
                o_ref[pl.ds(i, L)] = jnp.where(sel, m * ten, zf)

        pltpu.emit_pipeline(
            body, in_specs=_scan_specs(), out_specs=_scan_specs(), **_PIPE
        )(mse_hbm, out_hbm)

    return k(mse, *hists)


def kernel(yhat, y):
    mse, h1 = _pass_a(yhat.reshape(-1), y.reshape(-1))
    h2 = _hist_pass(mse, (h1,))
    h3 = _hist_pass(mse, (h1, h2))
    h4 = _hist_pass(mse, (h1, h2, h3))
    out = _pass_out(mse, (h1, h2, h3, h4))
    return out.reshape(yhat.shape)


# speedup vs baseline: 15.6338x; 1.0818x over previous
import dataclasses
import functools

import jax
import jax.numpy as jnp
from jax import lax
from jax.experimental import pallas as pl
from jax.experimental.pallas import tpu as pltpu
from jax.experimental.pallas import tpu_sc as plsc

N = 128 * 32768
K = int(N * 0.1)
L = 16
NW = 32
BLK = 8192
GRID = N // BLK
UNROLL = 8

_mesh = functools.partial(
    plsc.VectorSubcoreMesh, core_axis_name="core", subcore_axis_name="subcore"
)


def _cparams():
    cp = pltpu.CompilerParams()
    if "needs_layout_passes" in pltpu.CompilerParams.__dataclass_fields__:
        cp = dataclasses.replace(cp, needs_layout_passes=False)
    return cp


def _bcast(x, dtype=jnp.int32):
    return lax.broadcast_in_dim(jnp.asarray(x, dtype), (L,), ())


def _zero_hist(hist_ref, nbins):
    zeros = jnp.zeros((L,), jnp.int32)

    @pl.loop(0, nbins, step=L)
    def _(i):
        hist_ref[pl.ds(i, L)] = zeros


def _reduce_hist(h_hbm, tmp_ref, hsum_ref, nbins):
    pltpu.sync_copy(h_hbm, tmp_ref)

    @pl.loop(0, nbins, step=L)
    def _(c):
        def body(t, acc):
            return acc + tmp_ref[t, pl.ds(c, L)]

        hsum_ref[pl.ds(c, L)] = lax.fori_loop(
            0, NW, body, jnp.zeros((L,), jnp.int32)
        )


def _find(hsum_ref, r, nbins):
    nchunks = nbins // L

    def body(i, carry):
        s, csel, hsel, sbefore = carry
        c = nchunks - 1 - i
        h = hsum_ref[pl.ds(c * L, L)]
        t = jnp.sum(h)
        hit = jnp.logical_and(s < r, s + t >= r)
        hitv = lax.broadcast_in_dim(hit, (L,), ())
        csel = jnp.where(hit, c, csel)
        hsel = jnp.where(hitv, h, hsel)
        sbefore = jnp.where(hit, s, sbefore)
        return s + t, csel, hsel, sbefore

    zero = jnp.asarray(0, jnp.int32)
    _, csel, hsel, sbefore = lax.fori_loop(
        0, nchunks, body, (zero, zero, jnp.zeros((L,), jnp.int32), zero)
    )
    cnt_desc = lax.rev(hsel, (0,))
    cum = jnp.cumsum(cnt_desc)
    r_in = r - sbefore
    i_star = jnp.sum((cum < r_in).astype(jnp.int32))
    at = lax.iota(jnp.int32, L) == lax.broadcast_in_dim(i_star, (L,), ())
    cnt_at = jnp.sum(jnp.where(at, cnt_desc, 0))
    cum_before = jnp.sum(jnp.where(at, cum, 0)) - cnt_at
    bucket = csel * L + (L - 1 - i_star)
    return bucket, r_in - cum_before


_LEVELS = ((23, 256), (15, 256), (7, 256), (0, 128))


def _threshold_prefix(h_hbms, tmp_refs, hsum_ref):
    r = jnp.asarray(K, jnp.int32)
    prefix = jnp.asarray(0, jnp.int32)
    for (_, nb), h_hbm, tmp_ref in zip(_LEVELS, h_hbms, tmp_refs):
        _reduce_hist(h_hbm, tmp_ref, hsum_ref, nb)
        b, r = _find(hsum_ref, r, nb)
        prefix = prefix * nb + b
    return prefix, r


def _wid():
    return lax.axis_index("subcore") * 2 + lax.axis_index("core")


def _scan_specs():
    return [pl.BlockSpec((BLK,), lambda i: (i,))]


_PIPE = dict(
    grid=(GRID,),
    core_axis_name=("core", "subcore"),
    dimension_semantics=(pltpu.PARALLEL,),
)


def _hist_scratch(nlevels):
    return [pltpu.VMEM((NW, nb), jnp.int32) for _, nb in _LEVELS[:nlevels]] + [
        pltpu.VMEM((256,), jnp.int32)
    ]


def _pass_a(yhat, y):
    @functools.partial(
        pl.kernel,
        out_type=(
            jax.ShapeDtypeStruct((N,), jnp.float32),
            jax.ShapeDtypeStruct((NW, 256), jnp.int32),
        ),
        mesh=_mesh(),
        scratch_types=[pltpu.VMEM((256,), jnp.int32)],
        compiler_params=_cparams(),
    )
    def k(a_hbm, b_hbm, mse_hbm, h_hbm, hist_ref):
        _zero_hist(hist_ref, 256)
        ones = jnp.ones((L,), jnp.int32)
        sh = _bcast(23)

        def body(a_ref, b_ref, m_ref):
            @pl.loop(0, BLK, step=L * UNROLL)
            def _(i):
                for u in range(UNROLL):
                    s = pl.ds(i + u * L, L)
                    d = a_ref[s] - b_ref[s]
                    m = d * d
                    m_ref[s] = m
                    idx = lax.shift_right_logical(plsc.bitcast(m, jnp.int32), sh)
                    plsc.addupdate_scatter(hist_ref, [idx], ones)

        pltpu.emit_pipeline(
            body, in_specs=_scan_specs() * 2, out_specs=_scan_specs(), **_PIPE
        )(a_hbm, b_hbm, mse_hbm)
        pltpu.sync_copy(hist_ref, h_hbm.at[_wid()])

    return k(yhat, y)


def _hist_pass(mse, hists):
    lvl = len(hists)
    shift, nbins = _LEVELS[lvl]

    @functools.partial(
        pl.kernel,
        out_type=jax.ShapeDtypeStruct((NW, nbins), jnp.int32),
        mesh=_mesh(),
        scratch_types=_hist_scratch(lvl) + [pltpu.VMEM((nbins,), jnp.int32)],
        compiler_params=_cparams(),
    )
    def k(mse_hbm, *refs):
        h_hbms = refs[:lvl]
        out_hbm = refs[lvl]
        tmp_refs = refs[lvl + 1 : 2 * lvl + 1]
        hsum_ref = refs[2 * lvl + 1]
        hist_ref = refs[2 * lvl + 2]

        prefix, _ = _threshold_prefix(h_hbms, tmp_refs, hsum_ref)

        _zero_hist(hist_ref, nbins)
        ones = jnp.ones((L,), jnp.int32)
        shv = _bcast(shift)
        nbits = 8 if nbins == 256 else 7
        cshv = _bcast(shift + nbits)
        maskv = _bcast(nbins - 1)
        prefv = lax.broadcast_in_dim(prefix, (L,), ())

        def body(m_ref):
            @pl.loop(0, BLK, step=L * UNROLL)
            def _(i):
                for u in range(UNROLL):
                    pat = plsc.bitcast(m_ref[pl.ds(i + u * L, L)], jnp.int32)
                    cond = lax.shift_right_logical(pat, cshv) == prefv
                    idx = jnp.bitwise_and(
                        lax.shift_right_logical(pat, shv), maskv
                    )
                    plsc.addupdate_scatter(hist_ref, [idx], ones, mask=cond)

        pltpu.emit_pipeline(body, in_specs=_scan_specs(), out_specs=[], **_PIPE)(
            mse_hbm
        )
        pltpu.sync_copy(hist_ref, out_hbm.at[_wid()])

    return k(mse, *hists)


def _pass_out(mse, hists):
    @functools.partial(
        pl.kernel,
        out_type=jax.ShapeDtypeStruct((N,), jnp.float32),
        mesh=_mesh(),
        scratch_types=_hist_scratch(4),
        compiler_params=_cparams(),
    )
    def k(mse_hbm, h1, h2, h3, h4, out_hbm, t1, t2, t3, t4, hsum_ref):
        thresh, _ = _threshold_prefix((h1, h2, h3, h4), (t1, t2, t3, t4), hsum_ref)

        tv = lax.broadcast_in_dim(thresh, (L,), ())
        ten = jnp.full((L,), 10.0, jnp.float32)
        zf = jnp.zeros((L,), jnp.float32)

        def body(m_ref, o_ref):
            @pl.loop(0, BLK, step=L * UNROLL)
            def _(i):
                for u in range(UNROLL):
                    s = pl.ds(i + u * L, L)
                    m = m_ref[s]
                    sel = plsc.bitcast(m, jnp.int32) >= tv
                    o_ref[s] = jnp.where(sel, m * ten, zf)

        pltpu.emit_pipeline(
            body, in_specs=_scan_specs(), out_specs=_scan_specs(), **_PIPE
        )(mse_hbm, out_hbm)

    return k(mse, *hists)


def kernel(yhat, y):
    mse, h1 = _pass_a(yhat.reshape(-1), y.reshape(-1))
    h2 = _hist_pass(mse, (h1,))
    h3 = _hist_pass(mse, (h1, h2))
    h4 = _hist_pass(mse, (h1, h2, h3))
    out = _pass_out(mse, (h1, h2, h3, h4))
    return out.reshape(yhat.shape)
